# baseline (speedup 1.0000x reference)
import functools

import jax
import jax.numpy as jnp
from jax import lax
from jax.experimental import pallas as pl
from jax.experimental.pallas import tpu as pltpu
from jax.experimental.pallas import tpu_sc as plsc

N_NODES = 10000
N_EDGES = 320000
D = 128
DW = 16

NC = 1
NS = 16
NW = NC * NS

CHUNK = 50
NCHUNK = 400
NBUF = 4
LAG = 2
IG = 5
NIG = NCHUNK // IG
SG = 20
NSG = NCHUNK // SG

ZROWS = N_NODES // NS

BLK = 1000


def _sc_agg_body(x_hbm, src_hbm, dst_hbm, zeros_hbm, zerosd_hbm, ones_hbm,
                 out_hbm, outd_hbm, src_idx, dst_idx, rows, ones_v, acc,
                 accd, *sems):
    gsem = sems[0:NBUF]
    ssem = sems[NBUF:2 * NBUF]
    isem = sems[2 * NBUF:2 * NBUF + 2]
    jsem = sems[2 * NBUF + 2:2 * NBUF + 4]
    sid = lax.axis_index("s")
    wid = sid * NC + lax.axis_index("c")

    pltpu.sync_copy(zeros_hbm, acc.at[pl.ds(sid * ZROWS, ZROWS)])
    pltpu.sync_copy(zerosd_hbm, accd.at[pl.ds(sid * ZROWS, ZROWS)])
    pltpu.sync_copy(ones_hbm, ones_v)

    def fetch_idx(gi, b):
        pltpu.async_copy(src_hbm.at[wid * NIG + gi], src_idx.at[b], isem[b])
        pltpu.async_copy(dst_hbm.at[wid * NIG + gi], dst_idx.at[b], jsem[b])

    def wait_idx(gi, b):
        pltpu.make_async_copy(src_hbm.at[wid * NIG + gi], src_idx.at[b],
                              isem[b]).wait()
        pltpu.make_async_copy(dst_hbm.at[wid * NIG + gi], dst_idx.at[b],
                              jsem[b]).wait()

    def start_gather(bi, slot, rb):
        pltpu.async_copy(x_hbm.at[src_idx.at[bi, slot]], rows.at[rb],
                         gsem[rb])

    def wait_gather(bi, slot, rb):
        pltpu.make_async_copy(x_hbm.at[src_idx.at[bi, slot]], rows.at[rb],
                              gsem[rb]).wait()

    def start_scatter(bi, slot, rb):
        pltpu.async_copy(rows.at[rb], acc.at[dst_idx.at[bi, slot]], ssem[rb],
                         add=True)
        pltpu.async_copy(ones_v, accd.at[dst_idx.at[bi, slot]], ssem[rb],
                         add=True)

    def wait_scatter(bi, slot, rb):
        pltpu.make_async_copy(rows.at[rb], acc.at[dst_idx.at[bi, slot]],
                              ssem[rb]).wait()
        pltpu.make_async_copy(ones_v, accd.at[dst_idx.at[bi, slot]],
                              ssem[rb]).wait()

    def pos(t):
        tm = t % SG
        return (tm // IG) % 2, tm % IG, t % NBUF

    def emit(t, gi_dyn, first_window):
        bi, slot, rb = pos(t)
        if t % IG == 0 and not (first_window and t == 0):
            wait_idx(gi_dyn, bi)
        if not (first_window and t < NBUF):
            wait_scatter(bi, slot, rb)
        start_gather(bi, slot, rb)
        if not (first_window and t < LAG):
            pbi, pslot, prb = pos((t - LAG) % SG)
            wait_gather(pbi, pslot, prb)
            start_scatter(pbi, pslot, prb)
        if t % IG == NBUF - 1:
            tgt = gi_dyn + 1
            if first_window and t < IG:
                tgt = None
            if tgt is not None:
                fetch_idx(jnp.minimum(tgt, NIG - 1), (t // IG + 1) % 2)

    fetch_idx(0, 0)
    wait_idx(0, 0)
    fetch_idx(1, 1)
    plsc.subcore_barrier()
    for t in range(SG):
        emit(t, t // IG, True)

    def sg_body(sg, carry):
        g0 = sg * (SG // IG)
        for t in range(SG):
            emit(t, g0 + t // IG, False)
        return carry

    lax.fori_loop(1, NSG, sg_body, 0)

    for c in range(NCHUNK - LAG, NCHUNK):
        bi, slot, rb = pos(c % SG)
        wait_gather(bi, slot, rb)
        start_scatter(bi, slot, rb)
    for c in range(NCHUNK - NBUF, NCHUNK):
        bi, slot, rb = pos(c % SG)
        wait_scatter(bi, slot, rb)
    wait_idx(NIG - 1, 0)

    plsc.subcore_barrier()

    r0 = sid * ZROWS
    pltpu.sync_copy(acc.at[pl.ds(r0, ZROWS)], out_hbm.at[pl.ds(r0, ZROWS)])
    pltpu.sync_copy(accd.at[pl.ds(r0, ZROWS)], outd_hbm.at[pl.ds(r0, ZROWS)])


_sc_agg = functools.partial(
    pl.kernel,
    out_type=(jax.ShapeDtypeStruct((N_NODES, D), jnp.float32),
              jax.ShapeDtypeStruct((N_NODES, DW), jnp.float32)),
    mesh=plsc.VectorSubcoreMesh(core_axis_name="c", subcore_axis_name="s",
                                num_cores=NC),
    scratch_types=[
        pltpu.VMEM((2, IG, CHUNK), jnp.int32),
        pltpu.VMEM((2, IG, CHUNK), jnp.int32),
        pltpu.VMEM((NBUF, CHUNK, D), jnp.float32),
        pltpu.VMEM((CHUNK, DW), jnp.float32),
        pltpu.VMEM_SHARED((N_NODES, D), jnp.float32),
        pltpu.VMEM_SHARED((N_NODES, DW), jnp.float32),
    ] + [pltpu.SemaphoreType.DMA] * (2 * NBUF + 4),
    compiler_params=pltpu.CompilerParams(use_tc_tiling_on_sc=False),
)(_sc_agg_body)


def _dot_t(a, w):
    return lax.dot_general(a, w, (((1,), (1,)), ((), ())),
                           preferred_element_type=jnp.float32)


def _tc_body(p0, pd, xr, w1, b1r, w2, b2r, ws, bsr, o):
    deg = pd[:, 0:1]
    neigh = p0[...] * (1.0 / jnp.maximum(deg, 1.0))
    h = jnp.maximum(_dot_t(neigh, w1[...]) + b1r[...], 0.0)
    h = _dot_t(h, w2[...]) + b2r[...]
    o[...] = h + _dot_t(xr[...], ws[...]) + bsr[...]


def _tc_mlp(partial, partial_deg, x, W1, b1r, W2, b2r, Wself, bsr):
    nblk = N_NODES // BLK
    return pl.pallas_call(
        _tc_body,
        grid=(nblk,),
        in_specs=[
            pl.BlockSpec((BLK, D), lambda i: (i, 0)),
            pl.BlockSpec((BLK, DW), lambda i: (i, 0)),
            pl.BlockSpec((BLK, D), lambda i: (i, 0)),
            pl.BlockSpec((D, D), lambda i: (0, 0)),
            pl.BlockSpec((1, D), lambda i: (0, 0)),
            pl.BlockSpec((D, D), lambda i: (0, 0)),
            pl.BlockSpec((1, D), lambda i: (0, 0)),
            pl.BlockSpec((D, D), lambda i: (0, 0)),
            pl.BlockSpec((1, D), lambda i: (0, 0)),
        ],
        out_specs=pl.BlockSpec((BLK, D), lambda i: (i, 0)),
        out_shape=jax.ShapeDtypeStruct((N_NODES, D), jnp.float32),
    )(partial, partial_deg, x, W1, b1r, W2, b2r, Wself, bsr)


def kernel(x, edge_index, W1, b1, W2, b2, Wself, bself):
    src_p = edge_index[0].astype(jnp.int32).reshape(NW * NIG, IG, CHUNK)
    dst_p = edge_index[1].astype(jnp.int32).reshape(NW * NIG, IG, CHUNK)
    zeros = jnp.zeros((ZROWS, D), jnp.float32)
    zerosd = jnp.zeros((ZROWS, DW), jnp.float32)
    ones = jnp.concatenate(
        [jnp.ones((CHUNK, 1), jnp.float32),
         jnp.zeros((CHUNK, DW - 1), jnp.float32)], axis=1)

    partial, partial_deg = _sc_agg(x, src_p, dst_p, zeros, zerosd, ones)
    return _tc_mlp(partial, partial_deg, x, W1, b1.reshape(1, D),
                   W2, b2.reshape(1, D), Wself, bself.reshape(1, D))


# speedup vs baseline: 1.0105x; 1.0105x over previous
import functools

import jax
import jax.numpy as jnp
from jax import lax
from jax.experimental import pallas as pl
from jax.experimental.pallas import tpu as pltpu
from jax.experimental.pallas import tpu_sc as plsc

N_NODES = 10000
N_EDGES = 320000
D = 128
DW = 16

NC = 1
NS = 16
NW = NC * NS

CHUNK = 100
NCHUNK = 200
NBUF = 2
LAG = 1
IG = 5
NIG = NCHUNK // IG
SG = 10
NSG = NCHUNK // SG

ZROWS = N_NODES // NS

BLK = 1000


def _sc_agg_body(x_hbm, src_hbm, dst_hbm, zeros_hbm, zerosd_hbm, ones_hbm,
                 out_hbm, outd_hbm, src_idx, dst_idx, rows, ones_v, acc,
                 accd, *sems):
    gsem = sems[0:NBUF]
    ssem = sems[NBUF:2 * NBUF]
    isem = sems[2 * NBUF:2 * NBUF + 2]
    jsem = sems[2 * NBUF + 2:2 * NBUF + 4]
    sid = lax.axis_index("s")
    wid = sid * NC + lax.axis_index("c")

    pltpu.sync_copy(zeros_hbm, acc.at[pl.ds(sid * ZROWS, ZROWS)])
    pltpu.sync_copy(zerosd_hbm, accd.at[pl.ds(sid * ZROWS, ZROWS)])
    pltpu.sync_copy(ones_hbm, ones_v)

    def fetch_idx(gi, b):
        pltpu.async_copy(src_hbm.at[wid * NIG + gi], src_idx.at[b], isem[b])
        pltpu.async_copy(dst_hbm.at[wid * NIG + gi], dst_idx.at[b], jsem[b])

    def wait_idx(gi, b):
        pltpu.make_async_copy(src_hbm.at[wid * NIG + gi], src_idx.at[b],
                              isem[b]).wait()
        pltpu.make_async_copy(dst_hbm.at[wid * NIG + gi], dst_idx.at[b],
                              jsem[b]).wait()

    def start_gather(bi, slot, rb):
        pltpu.async_copy(x_hbm.at[src_idx.at[bi, slot]], rows.at[rb],
                         gsem[rb])

    def wait_gather(bi, slot, rb):
        pltpu.make_async_copy(x_hbm.at[src_idx.at[bi, slot]], rows.at[rb],
                              gsem[rb]).wait()

    def start_scatter(bi, slot, rb):
        pltpu.async_copy(rows.at[rb], acc.at[dst_idx.at[bi, slot]], ssem[rb],
                         add=True)
        pltpu.async_copy(ones_v, accd.at[dst_idx.at[bi, slot]], ssem[rb],
                         add=True)

    def wait_scatter(bi, slot, rb):
        pltpu.make_async_copy(rows.at[rb], acc.at[dst_idx.at[bi, slot]],
                              ssem[rb]).wait()
        pltpu.make_async_copy(ones_v, accd.at[dst_idx.at[bi, slot]],
                              ssem[rb]).wait()

    def pos(t):
        tm = t % SG
        return (tm // IG) % 2, tm % IG, t % NBUF

    def emit(t, gi_dyn, first_window):
        bi, slot, rb = pos(t)
        if t % IG == 0 and not (first_window and t == 0):
            wait_idx(gi_dyn, bi)
        if not (first_window and t < NBUF):
            wait_scatter(bi, slot, rb)
        start_gather(bi, slot, rb)
        if not (first_window and t < LAG):
            pbi, pslot, prb = pos((t - LAG) % SG)
            wait_gather(pbi, pslot, prb)
            start_scatter(pbi, pslot, prb)
        if t % IG == NBUF - 1:
            tgt = gi_dyn + 1
            if first_window and t < IG:
                tgt = None
            if tgt is not None:
                fetch_idx(jnp.minimum(tgt, NIG - 1), (t // IG + 1) % 2)

    fetch_idx(0, 0)
    wait_idx(0, 0)
    fetch_idx(1, 1)
    plsc.subcore_barrier()
    for t in range(SG):
        emit(t, t // IG, True)

    def sg_body(sg, carry):
        g0 = sg * (SG // IG)
        for t in range(SG):
            emit(t, g0 + t // IG, False)
        return carry

    lax.fori_loop(1, NSG, sg_body, 0)

    for c in range(NCHUNK - LAG, NCHUNK):
        bi, slot, rb = pos(c % SG)
        wait_gather(bi, slot, rb)
        start_scatter(bi, slot, rb)
    for c in range(NCHUNK - NBUF, NCHUNK):
        bi, slot, rb = pos(c % SG)
        wait_scatter(bi, slot, rb)
    wait_idx(NIG - 1, 0)

    plsc.subcore_barrier()

    r0 = sid * ZROWS
    pltpu.sync_copy(acc.at[pl.ds(r0, ZROWS)], out_hbm.at[pl.ds(r0, ZROWS)])
    pltpu.sync_copy(accd.at[pl.ds(r0, ZROWS)], outd_hbm.at[pl.ds(r0, ZROWS)])


_sc_agg = functools.partial(
    pl.kernel,
    out_type=(jax.ShapeDtypeStruct((N_NODES, D), jnp.float32),
              jax.ShapeDtypeStruct((N_NODES, DW), jnp.float32)),
    mesh=plsc.VectorSubcoreMesh(core_axis_name="c", subcore_axis_name="s",
                                num_cores=NC),
    scratch_types=[
        pltpu.VMEM((2, IG, CHUNK), jnp.int32),
        pltpu.VMEM((2, IG, CHUNK), jnp.int32),
        pltpu.VMEM((NBUF, CHUNK, D), jnp.float32),
        pltpu.VMEM((CHUNK, DW), jnp.float32),
        pltpu.VMEM_SHARED((N_NODES, D), jnp.float32),
        pltpu.VMEM_SHARED((N_NODES, DW), jnp.float32),
    ] + [pltpu.SemaphoreType.DMA] * (2 * NBUF + 4),
    compiler_params=pltpu.CompilerParams(use_tc_tiling_on_sc=False),
)(_sc_agg_body)


def _dot_t(a, w):
    return lax.dot_general(a, w, (((1,), (1,)), ((), ())),
                           preferred_element_type=jnp.float32)


def _tc_self_body(xr, ws, bsr, o):
    o[...] = _dot_t(xr[...], ws[...]) + bsr[...]


def _tc_self(x, Wself, bsr):
    nblk = N_NODES // BLK
    return pl.pallas_call(
        _tc_self_body,
        grid=(nblk,),
        in_specs=[
            pl.BlockSpec((BLK, D), lambda i: (i, 0)),
            pl.BlockSpec((D, D), lambda i: (0, 0)),
            pl.BlockSpec((1, D), lambda i: (0, 0)),
        ],
        out_specs=pl.BlockSpec((BLK, D), lambda i: (i, 0)),
        out_shape=jax.ShapeDtypeStruct((N_NODES, D), jnp.float32),
    )(x, Wself, bsr)


def _tc_body(p0, pd, hs, w1, b1r, w2, b2r, o):
    deg = pd[:, 0:1]
    neigh = p0[...] * (1.0 / jnp.maximum(deg, 1.0))
    h = jnp.maximum(_dot_t(neigh, w1[...]) + b1r[...], 0.0)
    h = _dot_t(h, w2[...]) + b2r[...]
    o[...] = h + hs[...]


def _tc_mlp(partial, partial_deg, hself, W1, b1r, W2, b2r):
    nblk = N_NODES // BLK
    return pl.pallas_call(
        _tc_body,
        grid=(nblk,),
        in_specs=[
            pl.BlockSpec((BLK, D), lambda i: (i, 0)),
            pl.BlockSpec((BLK, DW), lambda i: (i, 0)),
            pl.BlockSpec((BLK, D), lambda i: (i, 0)),
            pl.BlockSpec((D, D), lambda i: (0, 0)),
            pl.BlockSpec((1, D), lambda i: (0, 0)),
            pl.BlockSpec((D, D), lambda i: (0, 0)),
            pl.BlockSpec((1, D), lambda i: (0, 0)),
        ],
        out_specs=pl.BlockSpec((BLK, D), lambda i: (i, 0)),
        out_shape=jax.ShapeDtypeStruct((N_NODES, D), jnp.float32),
    )(partial, partial_deg, hself, W1, b1r, W2, b2r)


def kernel(x, edge_index, W1, b1, W2, b2, Wself, bself):
    src_p = edge_index[0].astype(jnp.int32).reshape(NW * NIG, IG, CHUNK)
    dst_p = edge_index[1].astype(jnp.int32).reshape(NW * NIG, IG, CHUNK)
    zeros = jnp.zeros((ZROWS, D), jnp.float32)
    zerosd = jnp.zeros((ZROWS, DW), jnp.float32)
    ones = jnp.concatenate(
        [jnp.ones((CHUNK, 1), jnp.float32),
         jnp.zeros((CHUNK, DW - 1), jnp.float32)], axis=1)

    hself = _tc_self(x, Wself, bself.reshape(1, D))
    partial, partial_deg = _sc_agg(x, src_p, dst_p, zeros, zerosd, ones)
    return _tc_mlp(partial, partial_deg, hself, W1, b1.reshape(1, D),
                   W2, b2.reshape(1, D))
